# initial kernel scaffold (unmeasured)
import jax
import jax.numpy as jnp
from jax import lax
from jax.experimental import pallas as pl
from jax.experimental.pallas import tpu as pltpu

S = 2048
K = 4096
N = 8192
NBLK = 1024
NSTEPS = N // NBLK
S_HALF = S // 2


def kernel(O, Wo):
    A = O.reshape(S, K)

    def body(a_ref, wo_ref, out_ref, send_buf, recv_buf, send_sems, recv_sems):
        j = pl.program_id(0)
        mx = lax.axis_index("x")
        my = lax.axis_index("y")
        mz = lax.axis_index("z")
        peer = (mx, my, 1 - mz)

        @pl.when(j == 0)
        def _():
            bsem = pltpu.get_barrier_semaphore()
            pl.semaphore_signal(
                bsem, inc=1, device_id=peer,
                device_id_type=pl.DeviceIdType.MESH,
            )
            pl.semaphore_wait(bsem, 1)

        keep = jnp.dot(
            a_ref[pl.ds(mz * S_HALF, S_HALF), :], wo_ref[:, :],
            preferred_element_type=jnp.float32,
        )
        send_buf[:, :] = jnp.dot(
            a_ref[pl.ds((1 - mz) * S_HALF, S_HALF), :], wo_ref[:, :],
            preferred_element_type=jnp.float32,
        )

        slot = lax.rem(j, 2)
        rdma = pltpu.make_async_remote_copy(
            src_ref=send_buf,
            dst_ref=recv_buf.at[slot],
            send_sem=send_sems.at[slot],
            recv_sem=recv_sems.at[slot],
            device_id=peer,
            device_id_type=pl.DeviceIdType.MESH,
        )
        rdma.start()
        rdma.wait()

        out_ref[:, :] = keep + recv_buf[slot, :, :]

    out = pl.pallas_call(
        body,
        grid=(NSTEPS,),
        in_specs=[
            pl.BlockSpec((S, K), lambda j: (0, 0)),
            pl.BlockSpec((K, NBLK), lambda j: (0, j)),
        ],
        out_specs=pl.BlockSpec((S_HALF, NBLK), lambda j: (0, j)),
        out_shape=jax.ShapeDtypeStruct((S_HALF, N), jnp.float32),
        scratch_shapes=[
            pltpu.VMEM((S_HALF, NBLK), jnp.float32),
            pltpu.VMEM((2, S_HALF, NBLK), jnp.float32),
            pltpu.SemaphoreType.DMA((2,)),
            pltpu.SemaphoreType.DMA((2,)),
        ],
        compiler_params=pltpu.CompilerParams(
            collective_id=0,
            dimension_semantics=("arbitrary",),
        ),
    )(A, Wo)

    return out.reshape(1, S_HALF, N)


# baseline (device time: 541330 ns/iter reference)
import jax
import jax.numpy as jnp
from jax import lax
from jax.experimental import pallas as pl
from jax.experimental.pallas import tpu as pltpu

S = 2048
K = 4096
N = 8192
NBLK = 512
NSTEPS = N // NBLK
S_HALF = S // 2


def kernel(O, Wo):
    A = O.reshape(S, K)

    def body(a_ref, wo_ref, out_ref, send_buf, recv_buf, send_sems, recv_sems):
        j = pl.program_id(0)
        mx = lax.axis_index("x")
        my = lax.axis_index("y")
        mz = lax.axis_index("z")
        peer = (mx, my, 1 - mz)

        @pl.when(j == 0)
        def _():
            bsem = pltpu.get_barrier_semaphore()
            pl.semaphore_signal(
                bsem, inc=1, device_id=peer,
                device_id_type=pl.DeviceIdType.MESH,
            )
            pl.semaphore_wait(bsem, 1)

        send_buf[:, :] = jnp.dot(
            a_ref[pl.ds((1 - mz) * S_HALF, S_HALF), :], wo_ref[:, :],
            preferred_element_type=jnp.float32,
        )

        slot = lax.rem(j, 2)
        rdma = pltpu.make_async_remote_copy(
            src_ref=send_buf,
            dst_ref=recv_buf.at[slot],
            send_sem=send_sems.at[slot],
            recv_sem=recv_sems.at[slot],
            device_id=peer,
            device_id_type=pl.DeviceIdType.MESH,
        )
        rdma.start()

        out_ref[:, :] = jnp.dot(
            a_ref[pl.ds(mz * S_HALF, S_HALF), :], wo_ref[:, :],
            preferred_element_type=jnp.float32,
        )

        rdma.wait()
        out_ref[:, :] += recv_buf[slot, :, :]

    out = pl.pallas_call(
        body,
        grid=(NSTEPS,),
        in_specs=[
            pl.BlockSpec((S, K), lambda j: (0, 0)),
            pl.BlockSpec((K, NBLK), lambda j: (0, j)),
        ],
        out_specs=pl.BlockSpec((S_HALF, NBLK), lambda j: (0, j)),
        out_shape=jax.ShapeDtypeStruct((S_HALF, N), jnp.float32),
        scratch_shapes=[
            pltpu.VMEM((S_HALF, NBLK), jnp.float32),
            pltpu.VMEM((2, S_HALF, NBLK), jnp.float32),
            pltpu.SemaphoreType.DMA((2,)),
            pltpu.SemaphoreType.DMA((2,)),
        ],
        compiler_params=pltpu.CompilerParams(
            collective_id=0,
            dimension_semantics=("arbitrary",),
            vmem_limit_bytes=100 * 1024 * 1024,
        ),
    )(A, Wo)

    return out.reshape(1, S_HALF, N)


# device time: 456109 ns/iter; 1.1868x vs baseline; 1.1868x over previous
import jax
import jax.numpy as jnp
from jax import lax
from jax.experimental import pallas as pl
from jax.experimental.pallas import tpu as pltpu

S = 2048
K = 4096
N = 8192
NBLK = 256
NB = N // NBLK
HALF = S // 2
Q = HALF // 4
NSLOT = 4

MESH = pl.DeviceIdType.MESH


def kernel(O, Wo):
    A = O.reshape(S, K)

    def body(a_ref, wo_ref, out_ref,
             zsend, zrecv, xy, yrecv,
             z_send_sems, z_recv_sems,
             x_send_sems, x_recv_sems,
             y_send_sems, y_recv_sems,
             zc, xc, yc):
        j = pl.program_id(0)
        mx = lax.axis_index("x")
        my = lax.axis_index("y")
        mz = lax.axis_index("z")
        zpeer = (mx, my, 1 - mz)
        xpeer = (1 - mx, my, mz)
        ypeer = (mx, 1 - my, mz)

        qi = 2 * mx + my
        qx = 2 * (1 - mx) + my
        qy1 = 2 * mx + (1 - my)
        qy2 = 2 * (1 - mx) + (1 - my)

        keep_base = mz * HALF
        send_base = (1 - mz) * HALF

        @pl.when(j == 0)
        def _():
            bsem = pltpu.get_barrier_semaphore()
            for nbr in (zpeer, xpeer, ypeer):
                pl.semaphore_signal(bsem, inc=1, device_id=nbr,
                                    device_id_type=MESH)
            pl.semaphore_wait(bsem, 3)

        @pl.when(j < NB)
        def _():
            b = j
            s2 = lax.rem(b, 2)
            s4 = lax.rem(b, NSLOT)

            @pl.when(b >= 2)
            def _():
                pltpu.make_async_remote_copy(
                    src_ref=zsend.at[s2], dst_ref=zrecv.at[s4],
                    send_sem=z_send_sems.at[s2], recv_sem=z_recv_sems.at[s4],
                    device_id=zpeer, device_id_type=MESH,
                ).wait_send()

            zsend[s2, :, :] = jnp.dot(
                a_ref[pl.ds(send_base + qi * Q, Q), :], wo_ref[:, :],
                preferred_element_type=jnp.float32,
            )

            @pl.when(b >= NSLOT)
            def _():
                pl.semaphore_wait(zc.at[s4], 1)

            pltpu.make_async_remote_copy(
                src_ref=zsend.at[s2], dst_ref=zrecv.at[s4],
                send_sem=z_send_sems.at[s2], recv_sem=z_recv_sems.at[s4],
                device_id=zpeer, device_id_type=MESH,
            ).start()

            @pl.when(b >= NSLOT)
            def _():
                pltpu.make_async_remote_copy(
                    src_ref=xy.at[s4, pl.ds(0, Q), :],
                    dst_ref=xy.at[s4, pl.ds(Q, Q), :],
                    send_sem=x_send_sems.at[s4], recv_sem=x_recv_sems.at[s4],
                    device_id=xpeer, device_id_type=MESH,
                ).wait_send()
                pltpu.make_async_remote_copy(
                    src_ref=xy.at[s4], dst_ref=yrecv.at[s4],
                    send_sem=y_send_sems.at[s4], recv_sem=y_recv_sems.at[s4],
                    device_id=ypeer, device_id_type=MESH,
                ).wait_send()
                pl.semaphore_signal(xc.at[s4], inc=1, device_id=xpeer,
                                    device_id_type=MESH)

            xy[s4, pl.ds(0, Q), :] = jnp.dot(
                a_ref[pl.ds(keep_base + qi * Q, Q), :], wo_ref[:, :],
                preferred_element_type=jnp.float32,
            )

        @pl.when(jnp.logical_and(j >= 1, j <= NB))
        def _():
            b = j - 1
            s4 = lax.rem(b, NSLOT)
            pltpu.make_async_remote_copy(
                src_ref=zsend.at[lax.rem(b, 2)], dst_ref=zrecv.at[s4],
                send_sem=z_send_sems.at[lax.rem(b, 2)],
                recv_sem=z_recv_sems.at[s4],
                device_id=zpeer, device_id_type=MESH,
            ).wait_recv()
            xy[s4, pl.ds(0, Q), :] += zrecv[s4, :, :]

            @pl.when(b <= NB - 1 - NSLOT)
            def _():
                pl.semaphore_signal(zc.at[s4], inc=1, device_id=zpeer,
                                    device_id_type=MESH)

            @pl.when(b >= NSLOT)
            def _():
                pl.semaphore_wait(xc.at[s4], 1)

            pltpu.make_async_remote_copy(
                src_ref=xy.at[s4, pl.ds(0, Q), :],
                dst_ref=xy.at[s4, pl.ds(Q, Q), :],
                send_sem=x_send_sems.at[s4], recv_sem=x_recv_sems.at[s4],
                device_id=xpeer, device_id_type=MESH,
            ).start()

        @pl.when(jnp.logical_and(j >= 2, j <= NB + 1))
        def _():
            b = j - 2
            s4 = lax.rem(b, NSLOT)
            pltpu.make_async_remote_copy(
                src_ref=xy.at[s4, pl.ds(0, Q), :],
                dst_ref=xy.at[s4, pl.ds(Q, Q), :],
                send_sem=x_send_sems.at[s4], recv_sem=x_recv_sems.at[s4],
                device_id=xpeer, device_id_type=MESH,
            ).wait_recv()

            @pl.when(b >= NSLOT)
            def _():
                pl.semaphore_wait(yc.at[s4], 1)

            pltpu.make_async_remote_copy(
                src_ref=xy.at[s4], dst_ref=yrecv.at[s4],
                send_sem=y_send_sems.at[s4], recv_sem=y_recv_sems.at[s4],
                device_id=ypeer, device_id_type=MESH,
            ).start()

        @pl.when(j >= 3)
        def _():
            b = j - 3
            s4 = lax.rem(b, NSLOT)
            pltpu.make_async_remote_copy(
                src_ref=xy.at[s4], dst_ref=yrecv.at[s4],
                send_sem=y_send_sems.at[s4], recv_sem=y_recv_sems.at[s4],
                device_id=ypeer, device_id_type=MESH,
            ).wait_recv()

            out_ref[pl.ds(qi * Q, Q), :] = xy[s4, pl.ds(0, Q), :]
            out_ref[pl.ds(qx * Q, Q), :] = xy[s4, pl.ds(Q, Q), :]
            out_ref[pl.ds(qy1 * Q, Q), :] = yrecv[s4, pl.ds(0, Q), :]
            out_ref[pl.ds(qy2 * Q, Q), :] = yrecv[s4, pl.ds(Q, Q), :]

            @pl.when(b <= NB - 1 - NSLOT)
            def _():
                pl.semaphore_signal(yc.at[s4], inc=1, device_id=ypeer,
                                    device_id_type=MESH)

        @pl.when(j == NB + 2)
        def _():
            for s in range(2):
                pltpu.make_async_remote_copy(
                    src_ref=zsend.at[s], dst_ref=zrecv.at[s],
                    send_sem=z_send_sems.at[s], recv_sem=z_recv_sems.at[s],
                    device_id=zpeer, device_id_type=MESH,
                ).wait_send()
            for s in range(NSLOT):
                pltpu.make_async_remote_copy(
                    src_ref=xy.at[s, pl.ds(0, Q), :],
                    dst_ref=xy.at[s, pl.ds(Q, Q), :],
                    send_sem=x_send_sems.at[s], recv_sem=x_recv_sems.at[s],
                    device_id=xpeer, device_id_type=MESH,
                ).wait_send()
                pltpu.make_async_remote_copy(
                    src_ref=xy.at[s], dst_ref=yrecv.at[s],
                    send_sem=y_send_sems.at[s], recv_sem=y_recv_sems.at[s],
                    device_id=ypeer, device_id_type=MESH,
                ).wait_send()

    out = pl.pallas_call(
        body,
        grid=(NB + 3,),
        in_specs=[
            pl.BlockSpec((S, K), lambda j: (0, 0)),
            pl.BlockSpec((K, NBLK), lambda j: (0, jnp.minimum(j, NB - 1))),
        ],
        out_specs=pl.BlockSpec(
            (HALF, NBLK), lambda j: (0, jnp.maximum(j - 3, 0))
        ),
        out_shape=jax.ShapeDtypeStruct((HALF, N), jnp.float32),
        scratch_shapes=[
            pltpu.VMEM((2, Q, NBLK), jnp.float32),
            pltpu.VMEM((NSLOT, Q, NBLK), jnp.float32),
            pltpu.VMEM((NSLOT, 2 * Q, NBLK), jnp.float32),
            pltpu.VMEM((NSLOT, 2 * Q, NBLK), jnp.float32),
            pltpu.SemaphoreType.DMA((2,)),
            pltpu.SemaphoreType.DMA((NSLOT,)),
            pltpu.SemaphoreType.DMA((NSLOT,)),
            pltpu.SemaphoreType.DMA((NSLOT,)),
            pltpu.SemaphoreType.DMA((NSLOT,)),
            pltpu.SemaphoreType.DMA((NSLOT,)),
            pltpu.SemaphoreType.REGULAR((NSLOT,)),
            pltpu.SemaphoreType.REGULAR((NSLOT,)),
            pltpu.SemaphoreType.REGULAR((NSLOT,)),
        ],
        compiler_params=pltpu.CompilerParams(
            collective_id=0,
            dimension_semantics=("arbitrary",),
            vmem_limit_bytes=100 * 1024 * 1024,
        ),
    )(A, Wo)

    return out.reshape(1, HALF, N)


# device time: 234208 ns/iter; 2.3113x vs baseline; 1.9475x over previous
import jax
import jax.numpy as jnp
from jax import lax
from jax.experimental import pallas as pl
from jax.experimental.pallas import tpu as pltpu

S = 2048
H = 32
D = 128
K = H * D
N = 8192
NBLK = 512
NB = N // NBLK
HALF = S // 2
Q = HALF // 4
NSLOT = 4

MESH = pl.DeviceIdType.MESH


def kernel(O, Wo):
    def body(o_hbm, wo_ref, out_ref,
             a2, zsend, zrecv, xy, yrecv,
             load_sems,
             z_send_sems, z_recv_sems,
             x_send_sems, x_recv_sems,
             y_send_sems, y_recv_sems,
             zc, xc, yc):
        j = pl.program_id(0)
        mx = lax.axis_index("x")
        my = lax.axis_index("y")
        mz = lax.axis_index("z")
        zpeer = (mx, my, 1 - mz)
        xpeer = (1 - mx, my, mz)
        ypeer = (mx, 1 - my, mz)

        qi = 2 * mx + my
        qx = 2 * (1 - mx) + my
        qy1 = 2 * mx + (1 - my)
        qy2 = 2 * (1 - mx) + (1 - my)

        keep_start = mz * HALF + qi * Q
        send_start = (1 - mz) * HALF + qi * Q

        @pl.when(j == 0)
        def _():
            for i, start in ((0, send_start), (1, keep_start)):
                for h in range(H):
                    pltpu.make_async_copy(
                        o_hbm.at[0, pl.ds(start, Q), h, :],
                        a2.at[i, :, pl.ds(h * D, D)],
                        load_sems.at[i],
                    ).start()
            bsem = pltpu.get_barrier_semaphore()
            for nbr in (zpeer, xpeer, ypeer):
                pl.semaphore_signal(bsem, inc=1, device_id=nbr,
                                    device_id_type=MESH)
            pl.semaphore_wait(bsem, 3)
            for i, start in ((0, send_start), (1, keep_start)):
                for h in range(H):
                    pltpu.make_async_copy(
                        o_hbm.at[0, pl.ds(start, Q), h, :],
                        a2.at[i, :, pl.ds(h * D, D)],
                        load_sems.at[i],
                    ).wait()

        @pl.when(j < NB)
        def _():
            b = j
            s2 = lax.rem(b, 2)
            s4 = lax.rem(b, NSLOT)

            @pl.when(b >= 2)
            def _():
                pltpu.make_async_remote_copy(
                    src_ref=zsend.at[s2], dst_ref=zrecv.at[s4],
                    send_sem=z_send_sems.at[s2], recv_sem=z_recv_sems.at[s4],
                    device_id=zpeer, device_id_type=MESH,
                ).wait_send()

            zsend[s2, :, :] = jnp.dot(
                a2[0, :, :], wo_ref[:, :],
                preferred_element_type=jnp.float32,
            )

            @pl.when(b >= NSLOT)
            def _():
                pl.semaphore_wait(zc.at[s4], 1)

            pltpu.make_async_remote_copy(
                src_ref=zsend.at[s2], dst_ref=zrecv.at[s4],
                send_sem=z_send_sems.at[s2], recv_sem=z_recv_sems.at[s4],
                device_id=zpeer, device_id_type=MESH,
            ).start()

            @pl.when(b >= NSLOT)
            def _():
                pltpu.make_async_remote_copy(
                    src_ref=xy.at[s4, pl.ds(0, Q), :],
                    dst_ref=xy.at[s4, pl.ds(Q, Q), :],
                    send_sem=x_send_sems.at[s4], recv_sem=x_recv_sems.at[s4],
                    device_id=xpeer, device_id_type=MESH,
                ).wait_send()
                pltpu.make_async_remote_copy(
                    src_ref=xy.at[s4], dst_ref=yrecv.at[s4],
                    send_sem=y_send_sems.at[s4], recv_sem=y_recv_sems.at[s4],
                    device_id=ypeer, device_id_type=MESH,
                ).wait_send()
                pl.semaphore_signal(xc.at[s4], inc=1, device_id=xpeer,
                                    device_id_type=MESH)

            xy[s4, pl.ds(0, Q), :] = jnp.dot(
                a2[1, :, :], wo_ref[:, :],
                preferred_element_type=jnp.float32,
            )

        @pl.when(jnp.logical_and(j >= 1, j <= NB))
        def _():
            b = j - 1
            s4 = lax.rem(b, NSLOT)
            pltpu.make_async_remote_copy(
                src_ref=zsend.at[lax.rem(b, 2)], dst_ref=zrecv.at[s4],
                send_sem=z_send_sems.at[lax.rem(b, 2)],
                recv_sem=z_recv_sems.at[s4],
                device_id=zpeer, device_id_type=MESH,
            ).wait_recv()
            xy[s4, pl.ds(0, Q), :] += zrecv[s4, :, :]

            @pl.when(b <= NB - 1 - NSLOT)
            def _():
                pl.semaphore_signal(zc.at[s4], inc=1, device_id=zpeer,
                                    device_id_type=MESH)

            @pl.when(b >= NSLOT)
            def _():
                pl.semaphore_wait(xc.at[s4], 1)

            pltpu.make_async_remote_copy(
                src_ref=xy.at[s4, pl.ds(0, Q), :],
                dst_ref=xy.at[s4, pl.ds(Q, Q), :],
                send_sem=x_send_sems.at[s4], recv_sem=x_recv_sems.at[s4],
                device_id=xpeer, device_id_type=MESH,
            ).start()

        @pl.when(jnp.logical_and(j >= 2, j <= NB + 1))
        def _():
            b = j - 2
            s4 = lax.rem(b, NSLOT)
            pltpu.make_async_remote_copy(
                src_ref=xy.at[s4, pl.ds(0, Q), :],
                dst_ref=xy.at[s4, pl.ds(Q, Q), :],
                send_sem=x_send_sems.at[s4], recv_sem=x_recv_sems.at[s4],
                device_id=xpeer, device_id_type=MESH,
            ).wait_recv()

            @pl.when(b >= NSLOT)
            def _():
                pl.semaphore_wait(yc.at[s4], 1)

            pltpu.make_async_remote_copy(
                src_ref=xy.at[s4], dst_ref=yrecv.at[s4],
                send_sem=y_send_sems.at[s4], recv_sem=y_recv_sems.at[s4],
                device_id=ypeer, device_id_type=MESH,
            ).start()

        @pl.when(j >= 3)
        def _():
            b = j - 3
            s4 = lax.rem(b, NSLOT)
            pltpu.make_async_remote_copy(
                src_ref=xy.at[s4], dst_ref=yrecv.at[s4],
                send_sem=y_send_sems.at[s4], recv_sem=y_recv_sems.at[s4],
                device_id=ypeer, device_id_type=MESH,
            ).wait_recv()

            out_ref[pl.ds(qi * Q, Q), :] = xy[s4, pl.ds(0, Q), :]
            out_ref[pl.ds(qx * Q, Q), :] = xy[s4, pl.ds(Q, Q), :]
            out_ref[pl.ds(qy1 * Q, Q), :] = yrecv[s4, pl.ds(0, Q), :]
            out_ref[pl.ds(qy2 * Q, Q), :] = yrecv[s4, pl.ds(Q, Q), :]

            @pl.when(b <= NB - 1 - NSLOT)
            def _():
                pl.semaphore_signal(yc.at[s4], inc=1, device_id=ypeer,
                                    device_id_type=MESH)

        @pl.when(j == NB + 2)
        def _():
            for s in range(2):
                pltpu.make_async_remote_copy(
                    src_ref=zsend.at[s], dst_ref=zrecv.at[s],
                    send_sem=z_send_sems.at[s], recv_sem=z_recv_sems.at[s],
                    device_id=zpeer, device_id_type=MESH,
                ).wait_send()
            for s in range(NSLOT):
                pltpu.make_async_remote_copy(
                    src_ref=xy.at[s, pl.ds(0, Q), :],
                    dst_ref=xy.at[s, pl.ds(Q, Q), :],
                    send_sem=x_send_sems.at[s], recv_sem=x_recv_sems.at[s],
                    device_id=xpeer, device_id_type=MESH,
                ).wait_send()
                pltpu.make_async_remote_copy(
                    src_ref=xy.at[s], dst_ref=yrecv.at[s],
                    send_sem=y_send_sems.at[s], recv_sem=y_recv_sems.at[s],
                    device_id=ypeer, device_id_type=MESH,
                ).wait_send()

    out = pl.pallas_call(
        body,
        grid=(NB + 3,),
        in_specs=[
            pl.BlockSpec(memory_space=pl.ANY),
            pl.BlockSpec((K, NBLK), lambda j: (0, jnp.minimum(j, NB - 1))),
        ],
        out_specs=pl.BlockSpec(
            (HALF, NBLK), lambda j: (0, jnp.maximum(j - 3, 0))
        ),
        out_shape=jax.ShapeDtypeStruct((HALF, N), jnp.float32),
        scratch_shapes=[
            pltpu.VMEM((2, Q, K), jnp.float32),
            pltpu.VMEM((2, Q, NBLK), jnp.float32),
            pltpu.VMEM((NSLOT, Q, NBLK), jnp.float32),
            pltpu.VMEM((NSLOT, 2 * Q, NBLK), jnp.float32),
            pltpu.VMEM((NSLOT, 2 * Q, NBLK), jnp.float32),
            pltpu.SemaphoreType.DMA((2,)),
            pltpu.SemaphoreType.DMA((2,)),
            pltpu.SemaphoreType.DMA((NSLOT,)),
            pltpu.SemaphoreType.DMA((NSLOT,)),
            pltpu.SemaphoreType.DMA((NSLOT,)),
            pltpu.SemaphoreType.DMA((NSLOT,)),
            pltpu.SemaphoreType.DMA((NSLOT,)),
            pltpu.SemaphoreType.REGULAR((NSLOT,)),
            pltpu.SemaphoreType.REGULAR((NSLOT,)),
            pltpu.SemaphoreType.REGULAR((NSLOT,)),
        ],
        compiler_params=pltpu.CompilerParams(
            collective_id=0,
            dimension_semantics=("arbitrary",),
            vmem_limit_bytes=100 * 1024 * 1024,
        ),
    )(O, Wo)

    return out.reshape(1, HALF, N)


# device time: 181826 ns/iter; 2.9772x vs baseline; 1.2881x over previous
import jax
import jax.numpy as jnp
from jax import lax
from jax.experimental import pallas as pl
from jax.experimental.pallas import tpu as pltpu

S = 2048
H = 32
D = 128
K = H * D
N = 8192
NBLK = 512
HB = NBLK // 2
NB = N // NBLK
HALF = S // 2
Q = HALF // 4
NSLOT = 4

MESH = pl.DeviceIdType.MESH


def kernel(O, Wo):
    def body(o_hbm, wo_ref, out_ref,
             a2, zsend, zrecv, mineq, qxbuf, qy1buf, qy2buf,
             load_sems,
             z_send_sems, z_recv_sems,
             h1x_send, h1x_recv, h1y_send, h1y_recv,
             h2y_send, h2y_recv, h2x_send, h2x_recv,
             zc, c_h1x, c_h1y, c_h2y, c_h2x):
        j = pl.program_id(0)
        mx = lax.axis_index("x")
        my = lax.axis_index("y")
        mz = lax.axis_index("z")
        zpeer = (mx, my, 1 - mz)
        xpeer = (1 - mx, my, mz)
        ypeer = (mx, 1 - my, mz)

        qi = 2 * mx + my
        qx = 2 * (1 - mx) + my
        qy1 = 2 * mx + (1 - my)
        qy2 = 2 * (1 - mx) + (1 - my)

        keep_start = mz * HALF + qi * Q
        send_start = (1 - mz) * HALF + qi * Q

        def z_desc(s2, s4):
            return pltpu.make_async_remote_copy(
                src_ref=zsend.at[s2], dst_ref=zrecv.at[s4],
                send_sem=z_send_sems.at[s2], recv_sem=z_recv_sems.at[s4],
                device_id=zpeer, device_id_type=MESH,
            )

        def h1x_desc(s4):
            return pltpu.make_async_remote_copy(
                src_ref=mineq.at[s4], dst_ref=qxbuf.at[s4],
                send_sem=h1x_send.at[s4], recv_sem=h1x_recv.at[s4],
                device_id=xpeer, device_id_type=MESH,
            )

        def h1y_desc(s4):
            return pltpu.make_async_remote_copy(
                src_ref=mineq.at[s4], dst_ref=qy1buf.at[s4],
                send_sem=h1y_send.at[s4], recv_sem=h1y_recv.at[s4],
                device_id=ypeer, device_id_type=MESH,
            )

        def h2y_desc(s4):
            return pltpu.make_async_remote_copy(
                src_ref=qxbuf.at[s4, :, pl.ds(0, HB)],
                dst_ref=qy2buf.at[s4, :, pl.ds(0, HB)],
                send_sem=h2y_send.at[s4], recv_sem=h2y_recv.at[s4],
                device_id=ypeer, device_id_type=MESH,
            )

        def h2x_desc(s4):
            return pltpu.make_async_remote_copy(
                src_ref=qy1buf.at[s4, :, pl.ds(HB, HB)],
                dst_ref=qy2buf.at[s4, :, pl.ds(HB, HB)],
                send_sem=h2x_send.at[s4], recv_sem=h2x_recv.at[s4],
                device_id=xpeer, device_id_type=MESH,
            )

        @pl.when(j == 0)
        def _():
            for i, start in ((0, send_start), (1, keep_start)):
                for h in range(H):
                    pltpu.make_async_copy(
                        o_hbm.at[0, pl.ds(start, Q), h, :],
                        a2.at[i, :, pl.ds(h * D, D)],
                        load_sems.at[i],
                    ).start()
            bsem = pltpu.get_barrier_semaphore()
            for nbr in (zpeer, xpeer, ypeer):
                pl.semaphore_signal(bsem, inc=1, device_id=nbr,
                                    device_id_type=MESH)
            pl.semaphore_wait(bsem, 3)
            for i, start in ((0, send_start), (1, keep_start)):
                for h in range(H):
                    pltpu.make_async_copy(
                        o_hbm.at[0, pl.ds(start, Q), h, :],
                        a2.at[i, :, pl.ds(h * D, D)],
                        load_sems.at[i],
                    ).wait()

        @pl.when(j < NB)
        def _():
            b = j
            s2 = lax.rem(b, 2)
            s4 = lax.rem(b, NSLOT)

            @pl.when(b >= 2)
            def _():
                z_desc(s2, s4).wait_send()

            zsend[s2, :, :] = jnp.dot(
                a2[0, :, :], wo_ref[:, :],
                preferred_element_type=jnp.float32,
            )

            @pl.when(b >= NSLOT)
            def _():
                pl.semaphore_wait(zc.at[s4], 1)

            z_desc(s2, s4).start()

            @pl.when(b >= NSLOT)
            def _():
                h1x_desc(s4).wait_send()
                h1y_desc(s4).wait_send()
                h2y_desc(s4).wait_send()
                h2x_desc(s4).wait_send()
                pl.semaphore_signal(c_h1x.at[s4], inc=1, device_id=xpeer,
                                    device_id_type=MESH)
                pl.semaphore_signal(c_h1y.at[s4], inc=1, device_id=ypeer,
                                    device_id_type=MESH)

            mineq[s4, :, :] = jnp.dot(
                a2[1, :, :], wo_ref[:, :],
                preferred_element_type=jnp.float32,
            )

        @pl.when(jnp.logical_and(j >= 1, j <= NB))
        def _():
            b = j - 1
            s4 = lax.rem(b, NSLOT)
            z_desc(lax.rem(b, 2), s4).wait_recv()
            mineq[s4, :, :] += zrecv[s4, :, :]

            @pl.when(b <= NB - 1 - NSLOT)
            def _():
                pl.semaphore_signal(zc.at[s4], inc=1, device_id=zpeer,
                                    device_id_type=MESH)

            @pl.when(b >= NSLOT)
            def _():
                pl.semaphore_wait(c_h1x.at[s4], 1)
                pl.semaphore_wait(c_h1y.at[s4], 1)

            h1x_desc(s4).start()
            h1y_desc(s4).start()

        @pl.when(jnp.logical_and(j >= 2, j <= NB + 1))
        def _():
            b = j - 2
            s4 = lax.rem(b, NSLOT)
            h1x_desc(s4).wait_recv()
            h1y_desc(s4).wait_recv()

            @pl.when(b >= NSLOT)
            def _():
                pl.semaphore_wait(c_h2y.at[s4], 1)
                pl.semaphore_wait(c_h2x.at[s4], 1)

            h2y_desc(s4).start()
            h2x_desc(s4).start()

        @pl.when(j >= 3)
        def _():
            b = j - 3
            s4 = lax.rem(b, NSLOT)
            h2y_desc(s4).wait_recv()
            h2x_desc(s4).wait_recv()

            out_ref[pl.ds(qi * Q, Q), :] = mineq[s4, :, :]
            out_ref[pl.ds(qx * Q, Q), :] = qxbuf[s4, :, :]
            out_ref[pl.ds(qy1 * Q, Q), :] = qy1buf[s4, :, :]
            out_ref[pl.ds(qy2 * Q, Q), :] = qy2buf[s4, :, :]

            @pl.when(b <= NB - 1 - NSLOT)
            def _():
                pl.semaphore_signal(c_h2y.at[s4], inc=1, device_id=ypeer,
                                    device_id_type=MESH)
                pl.semaphore_signal(c_h2x.at[s4], inc=1, device_id=xpeer,
                                    device_id_type=MESH)

        @pl.when(j == NB + 2)
        def _():
            for s in range(2):
                z_desc(s, s).wait_send()
            for s in range(NSLOT):
                h1x_desc(s).wait_send()
                h1y_desc(s).wait_send()
                h2y_desc(s).wait_send()
                h2x_desc(s).wait_send()

    out = pl.pallas_call(
        body,
        grid=(NB + 3,),
        in_specs=[
            pl.BlockSpec(memory_space=pl.ANY),
            pl.BlockSpec((K, NBLK), lambda j: (0, jnp.minimum(j, NB - 1))),
        ],
        out_specs=pl.BlockSpec(
            (HALF, NBLK), lambda j: (0, jnp.maximum(j - 3, 0))
        ),
        out_shape=jax.ShapeDtypeStruct((HALF, N), jnp.float32),
        scratch_shapes=[
            pltpu.VMEM((2, Q, K), jnp.float32),
            pltpu.VMEM((2, Q, NBLK), jnp.float32),
            pltpu.VMEM((NSLOT, Q, NBLK), jnp.float32),
            pltpu.VMEM((NSLOT, Q, NBLK), jnp.float32),
            pltpu.VMEM((NSLOT, Q, NBLK), jnp.float32),
            pltpu.VMEM((NSLOT, Q, NBLK), jnp.float32),
            pltpu.VMEM((NSLOT, Q, NBLK), jnp.float32),
            pltpu.SemaphoreType.DMA((2,)),
            pltpu.SemaphoreType.DMA((2,)),
            pltpu.SemaphoreType.DMA((NSLOT,)),
            pltpu.SemaphoreType.DMA((NSLOT,)),
            pltpu.SemaphoreType.DMA((NSLOT,)),
            pltpu.SemaphoreType.DMA((NSLOT,)),
            pltpu.SemaphoreType.DMA((NSLOT,)),
            pltpu.SemaphoreType.DMA((NSLOT,)),
            pltpu.SemaphoreType.DMA((NSLOT,)),
            pltpu.SemaphoreType.DMA((NSLOT,)),
            pltpu.SemaphoreType.DMA((NSLOT,)),
            pltpu.SemaphoreType.REGULAR((NSLOT,)),
            pltpu.SemaphoreType.REGULAR((NSLOT,)),
            pltpu.SemaphoreType.REGULAR((NSLOT,)),
            pltpu.SemaphoreType.REGULAR((NSLOT,)),
            pltpu.SemaphoreType.REGULAR((NSLOT,)),
        ],
        compiler_params=pltpu.CompilerParams(
            collective_id=0,
            dimension_semantics=("arbitrary",),
            vmem_limit_bytes=100 * 1024 * 1024,
        ),
    )(O, Wo)

    return out.reshape(1, HALF, N)


# device time: 181652 ns/iter; 2.9800x vs baseline; 1.0010x over previous
import jax
import jax.numpy as jnp
from jax import lax
from jax.experimental import pallas as pl
from jax.experimental.pallas import tpu as pltpu

S = 2048
H = 32
D = 128
K = H * D
N = 8192
NBLK = 512
HB = NBLK // 2
NB = N // NBLK
HALF = S // 2
Q = HALF // 4
NSLOT = 4

MESH = pl.DeviceIdType.MESH


def kernel(O, Wo):
    def body(o_hbm, wo_ref, out_ref,
             a2, zsend, zrecv, mineq, qxbuf, qy1buf, qy2buf,
             load_sems,
             z_send_sems, z_recv_sems,
             h1x_send, h1x_recv, h1y_send, h1y_recv,
             h2y_send, h2y_recv, h2x_send, h2x_recv,
             zc, c_h1x, c_h1y, c_h2y, c_h2x):
        j = pl.program_id(0)
        mx = lax.axis_index("x")
        my = lax.axis_index("y")
        mz = lax.axis_index("z")
        zpeer = (mx, my, 1 - mz)
        xpeer = (1 - mx, my, mz)
        ypeer = (mx, 1 - my, mz)

        qi = 2 * mx + my
        qx = 2 * (1 - mx) + my
        qy1 = 2 * mx + (1 - my)
        qy2 = 2 * (1 - mx) + (1 - my)

        keep_start = mz * HALF + qi * Q
        send_start = (1 - mz) * HALF + qi * Q

        def z_desc(s2, s4):
            return pltpu.make_async_remote_copy(
                src_ref=zsend.at[s2], dst_ref=zrecv.at[s4],
                send_sem=z_send_sems.at[s2], recv_sem=z_recv_sems.at[s4],
                device_id=zpeer, device_id_type=MESH,
            )

        def h1x_desc(s4):
            return pltpu.make_async_remote_copy(
                src_ref=mineq.at[s4], dst_ref=qxbuf.at[s4],
                send_sem=h1x_send.at[s4], recv_sem=h1x_recv.at[s4],
                device_id=xpeer, device_id_type=MESH,
            )

        def h1y_desc(s4):
            return pltpu.make_async_remote_copy(
                src_ref=mineq.at[s4], dst_ref=qy1buf.at[s4],
                send_sem=h1y_send.at[s4], recv_sem=h1y_recv.at[s4],
                device_id=ypeer, device_id_type=MESH,
            )

        def h2y_desc(s4):
            return pltpu.make_async_remote_copy(
                src_ref=qxbuf.at[s4, :, pl.ds(0, HB)],
                dst_ref=qy2buf.at[s4, :, pl.ds(0, HB)],
                send_sem=h2y_send.at[s4], recv_sem=h2y_recv.at[s4],
                device_id=ypeer, device_id_type=MESH,
            )

        def h2x_desc(s4):
            return pltpu.make_async_remote_copy(
                src_ref=qy1buf.at[s4, :, pl.ds(HB, HB)],
                dst_ref=qy2buf.at[s4, :, pl.ds(HB, HB)],
                send_sem=h2x_send.at[s4], recv_sem=h2x_recv.at[s4],
                device_id=xpeer, device_id_type=MESH,
            )

        @pl.when(j == 0)
        def _():
            for i, start in ((0, send_start), (1, keep_start)):
                for h in range(H):
                    pltpu.make_async_copy(
                        o_hbm.at[0, pl.ds(start, Q), h, :],
                        a2.at[i, :, pl.ds(h * D, D)],
                        load_sems.at[i],
                    ).start()
            bsem = pltpu.get_barrier_semaphore()
            for nbr in (zpeer, xpeer, ypeer):
                pl.semaphore_signal(bsem, inc=1, device_id=nbr,
                                    device_id_type=MESH)
            pl.semaphore_wait(bsem, 3)
            for i, start in ((0, send_start), (1, keep_start)):
                for h in range(H):
                    pltpu.make_async_copy(
                        o_hbm.at[0, pl.ds(start, Q), h, :],
                        a2.at[i, :, pl.ds(h * D, D)],
                        load_sems.at[i],
                    ).wait()

        @pl.when(j < NB)
        def _():
            b = j
            s2 = lax.rem(b, 2)
            s4 = lax.rem(b, NSLOT)

            @pl.when(b >= 2)
            def _():
                z_desc(s2, s4).wait_send()

            zsend[s2, :, :] = jnp.dot(
                a2[0, :, :], wo_ref[:, :],
                preferred_element_type=jnp.float32,
            )

            @pl.when(b >= NSLOT)
            def _():
                pl.semaphore_wait(zc.at[s4], 1)

            z_desc(s2, s4).start()

            @pl.when(b >= NSLOT)
            def _():
                h1x_desc(s4).wait_send()
                h1y_desc(s4).wait_send()
                h2y_desc(s4).wait_send()
                h2x_desc(s4).wait_send()
                pl.semaphore_signal(c_h1x.at[s4], inc=1, device_id=xpeer,
                                    device_id_type=MESH)
                pl.semaphore_signal(c_h1y.at[s4], inc=1, device_id=ypeer,
                                    device_id_type=MESH)

            mineq[s4, :, :] = jnp.dot(
                a2[1, :, :], wo_ref[:, :],
                preferred_element_type=jnp.float32,
            )

        @pl.when(jnp.logical_and(j >= 1, j <= NB))
        def _():
            b = j - 1
            s4 = lax.rem(b, NSLOT)
            z_desc(lax.rem(b, 2), s4).wait_recv()
            mineq[s4, :, :] += zrecv[s4, :, :]

            @pl.when(b <= NB - 1 - NSLOT)
            def _():
                pl.semaphore_signal(zc.at[s4], inc=1, device_id=zpeer,
                                    device_id_type=MESH)

            @pl.when(b >= NSLOT)
            def _():
                pl.semaphore_wait(c_h1x.at[s4], 1)
                pl.semaphore_wait(c_h1y.at[s4], 1)

            h1x_desc(s4).start()
            h1y_desc(s4).start()

        @pl.when(jnp.logical_and(j >= 2, j <= NB + 1))
        def _():
            b = j - 2
            s4 = lax.rem(b, NSLOT)
            h1x_desc(s4).wait_recv()
            h1y_desc(s4).wait_recv()

            @pl.when(b >= NSLOT)
            def _():
                pl.semaphore_wait(c_h2y.at[s4], 1)
                pl.semaphore_wait(c_h2x.at[s4], 1)

            h2y_desc(s4).start()
            h2x_desc(s4).start()

        @pl.when(j >= 3)
        def _():
            b = j - 3
            s4 = lax.rem(b, NSLOT)
            h2y_desc(s4).wait_recv()
            h2x_desc(s4).wait_recv()

            out_ref[0, pl.ds(qi * Q, Q), :] = mineq[s4, :, :]
            out_ref[0, pl.ds(qx * Q, Q), :] = qxbuf[s4, :, :]
            out_ref[0, pl.ds(qy1 * Q, Q), :] = qy1buf[s4, :, :]
            out_ref[0, pl.ds(qy2 * Q, Q), :] = qy2buf[s4, :, :]

            @pl.when(b <= NB - 1 - NSLOT)
            def _():
                pl.semaphore_signal(c_h2y.at[s4], inc=1, device_id=ypeer,
                                    device_id_type=MESH)
                pl.semaphore_signal(c_h2x.at[s4], inc=1, device_id=xpeer,
                                    device_id_type=MESH)

        @pl.when(j == NB + 2)
        def _():
            for s in range(2):
                z_desc(s, s).wait_send()
            for s in range(NSLOT):
                h1x_desc(s).wait_send()
                h1y_desc(s).wait_send()
                h2y_desc(s).wait_send()
                h2x_desc(s).wait_send()

    out = pl.pallas_call(
        body,
        grid=(NB + 3,),
        in_specs=[
            pl.BlockSpec(memory_space=pl.ANY),
            pl.BlockSpec((K, NBLK), lambda j: (0, jnp.minimum(j, NB - 1))),
        ],
        out_specs=pl.BlockSpec(
            (1, HALF, NBLK), lambda j: (0, 0, jnp.maximum(j - 3, 0))
        ),
        out_shape=jax.ShapeDtypeStruct((1, HALF, N), jnp.float32),
        scratch_shapes=[
            pltpu.VMEM((2, Q, K), jnp.float32),
            pltpu.VMEM((2, Q, NBLK), jnp.float32),
            pltpu.VMEM((NSLOT, Q, NBLK), jnp.float32),
            pltpu.VMEM((NSLOT, Q, NBLK), jnp.float32),
            pltpu.VMEM((NSLOT, Q, NBLK), jnp.float32),
            pltpu.VMEM((NSLOT, Q, NBLK), jnp.float32),
            pltpu.VMEM((NSLOT, Q, NBLK), jnp.float32),
            pltpu.SemaphoreType.DMA((2,)),
            pltpu.SemaphoreType.DMA((2,)),
            pltpu.SemaphoreType.DMA((NSLOT,)),
            pltpu.SemaphoreType.DMA((NSLOT,)),
            pltpu.SemaphoreType.DMA((NSLOT,)),
            pltpu.SemaphoreType.DMA((NSLOT,)),
            pltpu.SemaphoreType.DMA((NSLOT,)),
            pltpu.SemaphoreType.DMA((NSLOT,)),
            pltpu.SemaphoreType.DMA((NSLOT,)),
            pltpu.SemaphoreType.DMA((NSLOT,)),
            pltpu.SemaphoreType.DMA((NSLOT,)),
            pltpu.SemaphoreType.REGULAR((NSLOT,)),
            pltpu.SemaphoreType.REGULAR((NSLOT,)),
            pltpu.SemaphoreType.REGULAR((NSLOT,)),
            pltpu.SemaphoreType.REGULAR((NSLOT,)),
            pltpu.SemaphoreType.REGULAR((NSLOT,)),
        ],
        compiler_params=pltpu.CompilerParams(
            collective_id=0,
            dimension_semantics=("arbitrary",),
            vmem_limit_bytes=100 * 1024 * 1024,
        ),
    )(O, Wo)

    return out


# device time: 180396 ns/iter; 3.0008x vs baseline; 1.0070x over previous
import jax
import jax.numpy as jnp
from jax import lax
from jax.experimental import pallas as pl
from jax.experimental.pallas import tpu as pltpu

S = 2048
H = 32
D = 128
K = H * D
N = 8192
NBLK = 512
HB = NBLK // 2
NB = N // NBLK
HALF = S // 2
Q = HALF // 4
NSLOT = 4

MESH = pl.DeviceIdType.MESH


def kernel(O, Wo):
    def body(o_hbm, wo_ref, out_ref,
             a2, zsend, zrecv, mineq, qxbuf, qy1buf, qy2buf,
             load_sems,
             z_send_sems, z_recv_sems,
             h1x_send, h1x_recv, h1y_send, h1y_recv,
             h2y_send, h2y_recv, h2x_send, h2x_recv,
             zc, c_h1x, c_h1y, c_h2y, c_h2x):
        j = pl.program_id(0)
        mx = lax.axis_index("x")
        my = lax.axis_index("y")
        mz = lax.axis_index("z")
        zpeer = (mx, my, 1 - mz)
        xpeer = (1 - mx, my, mz)
        ypeer = (mx, 1 - my, mz)

        qi = 2 * mx + my
        qx = 2 * (1 - mx) + my
        qy1 = 2 * mx + (1 - my)
        qy2 = 2 * (1 - mx) + (1 - my)

        keep_start = mz * HALF + qi * Q
        send_start = (1 - mz) * HALF + qi * Q

        def z_desc(s2, s4):
            return pltpu.make_async_remote_copy(
                src_ref=zsend.at[s2], dst_ref=zrecv.at[s4],
                send_sem=z_send_sems.at[s2], recv_sem=z_recv_sems.at[s4],
                device_id=zpeer, device_id_type=MESH,
            )

        def h1x_desc(s4):
            return pltpu.make_async_remote_copy(
                src_ref=mineq.at[s4], dst_ref=qxbuf.at[s4],
                send_sem=h1x_send.at[s4], recv_sem=h1x_recv.at[s4],
                device_id=xpeer, device_id_type=MESH,
            )

        def h1y_desc(s4):
            return pltpu.make_async_remote_copy(
                src_ref=mineq.at[s4], dst_ref=qy1buf.at[s4],
                send_sem=h1y_send.at[s4], recv_sem=h1y_recv.at[s4],
                device_id=ypeer, device_id_type=MESH,
            )

        def h2y_desc(s4):
            return pltpu.make_async_remote_copy(
                src_ref=qxbuf.at[s4, :, pl.ds(0, HB)],
                dst_ref=qy2buf.at[s4, :, pl.ds(0, HB)],
                send_sem=h2y_send.at[s4], recv_sem=h2y_recv.at[s4],
                device_id=ypeer, device_id_type=MESH,
            )

        def h2x_desc(s4):
            return pltpu.make_async_remote_copy(
                src_ref=qy1buf.at[s4, :, pl.ds(HB, HB)],
                dst_ref=qy2buf.at[s4, :, pl.ds(HB, HB)],
                send_sem=h2x_send.at[s4], recv_sem=h2x_recv.at[s4],
                device_id=xpeer, device_id_type=MESH,
            )

        @pl.when(j == 0)
        def _():
            for i, start in ((0, send_start), (1, keep_start)):
                for h in range(H):
                    pltpu.make_async_copy(
                        o_hbm.at[0, pl.ds(start, Q), h, :],
                        a2.at[i, :, pl.ds(h * D, D)],
                        load_sems.at[i],
                    ).start()
            bsem = pltpu.get_barrier_semaphore()
            for nbr in (zpeer, xpeer, ypeer):
                pl.semaphore_signal(bsem, inc=1, device_id=nbr,
                                    device_id_type=MESH)
            pl.semaphore_wait(bsem, 3)
            for h in range(H):
                pltpu.make_async_copy(
                    o_hbm.at[0, pl.ds(send_start, Q), h, :],
                    a2.at[0, :, pl.ds(h * D, D)],
                    load_sems.at[0],
                ).wait()

        @pl.when(j < NB)
        def _():
            b = j
            s2 = lax.rem(b, 2)
            s4 = lax.rem(b, NSLOT)

            @pl.when(b >= 2)
            def _():
                z_desc(s2, s4).wait_send()

            zsend[s2, :, :] = jnp.dot(
                a2[0, :, :], wo_ref[:, :],
                preferred_element_type=jnp.float32,
            )

            @pl.when(b >= NSLOT)
            def _():
                pl.semaphore_wait(zc.at[s4], 1)

            z_desc(s2, s4).start()

            @pl.when(b == 0)
            def _():
                for h in range(H):
                    pltpu.make_async_copy(
                        o_hbm.at[0, pl.ds(keep_start, Q), h, :],
                        a2.at[1, :, pl.ds(h * D, D)],
                        load_sems.at[1],
                    ).wait()

            @pl.when(b >= NSLOT)
            def _():
                h1x_desc(s4).wait_send()
                h1y_desc(s4).wait_send()
                h2y_desc(s4).wait_send()
                h2x_desc(s4).wait_send()
                pl.semaphore_signal(c_h1x.at[s4], inc=1, device_id=xpeer,
                                    device_id_type=MESH)
                pl.semaphore_signal(c_h1y.at[s4], inc=1, device_id=ypeer,
                                    device_id_type=MESH)

            mineq[s4, :, :] = jnp.dot(
                a2[1, :, :], wo_ref[:, :],
                preferred_element_type=jnp.float32,
            )

        @pl.when(jnp.logical_and(j >= 1, j <= NB))
        def _():
            b = j - 1
            s4 = lax.rem(b, NSLOT)
            z_desc(lax.rem(b, 2), s4).wait_recv()
            mineq[s4, :, :] += zrecv[s4, :, :]

            @pl.when(b <= NB - 1 - NSLOT)
            def _():
                pl.semaphore_signal(zc.at[s4], inc=1, device_id=zpeer,
                                    device_id_type=MESH)

            @pl.when(b >= NSLOT)
            def _():
                pl.semaphore_wait(c_h1x.at[s4], 1)
                pl.semaphore_wait(c_h1y.at[s4], 1)

            h1x_desc(s4).start()
            h1y_desc(s4).start()

        @pl.when(jnp.logical_and(j >= 2, j <= NB + 1))
        def _():
            b = j - 2
            s4 = lax.rem(b, NSLOT)
            h1x_desc(s4).wait_recv()
            h1y_desc(s4).wait_recv()

            @pl.when(b >= NSLOT)
            def _():
                pl.semaphore_wait(c_h2y.at[s4], 1)
                pl.semaphore_wait(c_h2x.at[s4], 1)

            h2y_desc(s4).start()
            h2x_desc(s4).start()

        @pl.when(j >= 3)
        def _():
            b = j - 3
            s4 = lax.rem(b, NSLOT)
            h2y_desc(s4).wait_recv()
            h2x_desc(s4).wait_recv()

            out_ref[0, pl.ds(qi * Q, Q), :] = mineq[s4, :, :]
            out_ref[0, pl.ds(qx * Q, Q), :] = qxbuf[s4, :, :]
            out_ref[0, pl.ds(qy1 * Q, Q), :] = qy1buf[s4, :, :]
            out_ref[0, pl.ds(qy2 * Q, Q), :] = qy2buf[s4, :, :]

            @pl.when(b <= NB - 1 - NSLOT)
            def _():
                pl.semaphore_signal(c_h2y.at[s4], inc=1, device_id=ypeer,
                                    device_id_type=MESH)
                pl.semaphore_signal(c_h2x.at[s4], inc=1, device_id=xpeer,
                                    device_id_type=MESH)

        @pl.when(j == NB + 2)
        def _():
            for s in range(2):
                z_desc(s, s).wait_send()
            for s in range(NSLOT):
                h1x_desc(s).wait_send()
                h1y_desc(s).wait_send()
                h2y_desc(s).wait_send()
                h2x_desc(s).wait_send()

    out = pl.pallas_call(
        body,
        grid=(NB + 3,),
        in_specs=[
            pl.BlockSpec(memory_space=pl.ANY),
            pl.BlockSpec((K, NBLK), lambda j: (0, jnp.minimum(j, NB - 1))),
        ],
        out_specs=pl.BlockSpec(
            (1, HALF, NBLK), lambda j: (0, 0, jnp.maximum(j - 3, 0))
        ),
        out_shape=jax.ShapeDtypeStruct((1, HALF, N), jnp.float32),
        scratch_shapes=[
            pltpu.VMEM((2, Q, K), jnp.float32),
            pltpu.VMEM((2, Q, NBLK), jnp.float32),
            pltpu.VMEM((NSLOT, Q, NBLK), jnp.float32),
            pltpu.VMEM((NSLOT, Q, NBLK), jnp.float32),
            pltpu.VMEM((NSLOT, Q, NBLK), jnp.float32),
            pltpu.VMEM((NSLOT, Q, NBLK), jnp.float32),
            pltpu.VMEM((NSLOT, Q, NBLK), jnp.float32),
            pltpu.SemaphoreType.DMA((2,)),
            pltpu.SemaphoreType.DMA((2,)),
            pltpu.SemaphoreType.DMA((NSLOT,)),
            pltpu.SemaphoreType.DMA((NSLOT,)),
            pltpu.SemaphoreType.DMA((NSLOT,)),
            pltpu.SemaphoreType.DMA((NSLOT,)),
            pltpu.SemaphoreType.DMA((NSLOT,)),
            pltpu.SemaphoreType.DMA((NSLOT,)),
            pltpu.SemaphoreType.DMA((NSLOT,)),
            pltpu.SemaphoreType.DMA((NSLOT,)),
            pltpu.SemaphoreType.DMA((NSLOT,)),
            pltpu.SemaphoreType.REGULAR((NSLOT,)),
            pltpu.SemaphoreType.REGULAR((NSLOT,)),
            pltpu.SemaphoreType.REGULAR((NSLOT,)),
            pltpu.SemaphoreType.REGULAR((NSLOT,)),
            pltpu.SemaphoreType.REGULAR((NSLOT,)),
        ],
        compiler_params=pltpu.CompilerParams(
            collective_id=0,
            dimension_semantics=("arbitrary",),
            vmem_limit_bytes=100 * 1024 * 1024,
        ),
    )(O, Wo)

    return out
